# baseline (speedup 1.0000x reference)
import jax
import jax.numpy as jnp
from jax import lax
from jax.experimental import pallas as pl
from jax.experimental.pallas import tpu as pltpu
from jax.experimental.pallas import tpu_sc as plsc

N_NODES = 10000
N_EDGES = 160000
NC, NS, LANES = 2, 16, 16
NW = NC * NS
K = 128

NCH_MP = 42
PW_MP = NCH_MP * K
EP_PAD = NW * PW_MP

NCH_F = 40
PW_F = NCH_F * K
E_PAD = NW * PW_F

N_ACC = 10112
RPT = N_ACC // NS
TRASH = N_NODES + 8

_MESH = dict(core_axis_name="c", subcore_axis_name="s")
_SC_PARAMS = pltpu.CompilerParams(needs_layout_passes=False,
                                  use_tc_tiling_on_sc=False)


def _iota16():
    return lax.broadcasted_iota(jnp.int32, (LANES,), 0)


def _build_deg():
    def body(dstg, onesh, zrows, out, dst2d, onesb, dacc):
        c = lax.axis_index("c")
        s = lax.axis_index("s")
        wid = s * NC + c
        pltpu.sync_copy(dstg.at[wid], dst2d)
        pltpu.sync_copy(onesh, onesb)
        pltpu.sync_copy(zrows, dacc.at[pl.ds(s * RPT, RPT)])
        plsc.subcore_barrier()

        def chunk(i, carry):
            pltpu.sync_copy(onesb, dacc.at[dst2d.at[i]], add=True)
            return carry

        lax.fori_loop(0, NCH_MP, chunk, 0)
        plsc.subcore_barrier()
        pltpu.sync_copy(dacc.at[pl.ds(s * RPT, RPT)],
                        out.at[c, pl.ds(s * RPT, RPT)])

    return pl.kernel(
        body,
        out_type=jax.ShapeDtypeStruct((NC, N_ACC, 16), jnp.float32),
        mesh=plsc.VectorSubcoreMesh(**_MESH),
        compiler_params=_SC_PARAMS,
        scratch_types=[
            pltpu.VMEM((NCH_MP, K), jnp.int32),
            pltpu.VMEM((K, 16), jnp.float32),
            pltpu.VMEM_SHARED((N_ACC, 16), jnp.float32),
        ],
    )


def _build_norm():
    def body(srcg, dstg, dinvh, out, src2d, dst2d, dinvv, norm2d):
        c = lax.axis_index("c")
        s = lax.axis_index("s")
        wid = s * NC + c
        pltpu.sync_copy(srcg.at[wid], src2d)
        pltpu.sync_copy(dstg.at[wid], dst2d)
        pltpu.sync_copy(dinvh, dinvv)

        def chunk(i, carry):
            for j in range(K // LANES):
                s16 = src2d[i, pl.ds(j * LANES, LANES)]
                d16 = dst2d[i, pl.ds(j * LANES, LANES)]
                a = plsc.load_gather(dinvv, [s16])
                b = plsc.load_gather(dinvv, [d16])
                norm2d[i, pl.ds(j * LANES, LANES)] = a * b
            return carry

        lax.fori_loop(0, NCH_MP, chunk, 0)
        pltpu.sync_copy(norm2d, out.at[wid])

    return pl.kernel(
        body,
        out_type=jax.ShapeDtypeStruct((NW, NCH_MP, K), jnp.float32),
        mesh=plsc.VectorSubcoreMesh(**_MESH),
        compiler_params=_SC_PARAMS,
        scratch_types=[
            pltpu.VMEM((NCH_MP, K), jnp.int32),
            pltpu.VMEM((NCH_MP, K), jnp.int32),
            pltpu.VMEM((N_ACC,), jnp.float32),
            pltpu.VMEM((NCH_MP, K), jnp.float32),
        ],
    )


def _build_wgt():
    def body(srcg, dstg, ash, adh, out, src2d, dst2d, asv, adv, w2d):
        c = lax.axis_index("c")
        s = lax.axis_index("s")
        wid = s * NC + c
        pltpu.sync_copy(srcg.at[wid], src2d)
        pltpu.sync_copy(dstg.at[wid], dst2d)
        pltpu.sync_copy(ash, asv)
        pltpu.sync_copy(adh, adv)

        def chunk(i, carry):
            for j in range(K // LANES):
                s16 = src2d[i, pl.ds(j * LANES, LANES)]
                d16 = dst2d[i, pl.ds(j * LANES, LANES)]
                e = (plsc.load_gather(asv, [s16])
                     + plsc.load_gather(adv, [d16]))
                e = jnp.maximum(e, e * 0.2)
                w2d[i, pl.ds(j * LANES, LANES)] = jnp.exp(e)
            return carry

        lax.fori_loop(0, NCH_MP, chunk, 0)
        pltpu.sync_copy(w2d, out.at[wid])

    return pl.kernel(
        body,
        out_type=jax.ShapeDtypeStruct((NW, NCH_MP, K), jnp.float32),
        mesh=plsc.VectorSubcoreMesh(**_MESH),
        compiler_params=_SC_PARAMS,
        scratch_types=[
            pltpu.VMEM((NCH_MP, K), jnp.int32),
            pltpu.VMEM((NCH_MP, K), jnp.int32),
            pltpu.VMEM((N_ACC,), jnp.float32),
            pltpu.VMEM((N_ACC,), jnp.float32),
            pltpu.VMEM((NCH_MP, K), jnp.float32),
        ],
    )


def _build_gcn(D):
    G = D // LANES

    def body(h, srcg, dstg, normg, zrows, out, src2d, dst2d, norm2d, rows,
             acc, sem):
        c = lax.axis_index("c")
        s = lax.axis_index("s")
        wid = s * NC + c
        pltpu.sync_copy(srcg.at[wid], src2d)
        pltpu.sync_copy(dstg.at[wid], dst2d)
        pltpu.sync_copy(normg.at[wid], norm2d)
        pltpu.sync_copy(zrows, acc.at[pl.ds(s * RPT, RPT)])
        plsc.subcore_barrier()

        iota = _iota16()
        cols = [jnp.full((LANES,), cidx, jnp.int32) for cidx in range(D)]

        def chunk(i, carry):
            pltpu.async_copy(h.at[src2d.at[i]], rows, sem).wait()

            def grp(j, carry2):
                nv = norm2d[i, pl.ds(j * LANES, LANES)]
                r16 = j * LANES + iota
                for cidx in range(D):
                    v = plsc.load_gather(rows, [r16, cols[cidx]])
                    plsc.store_scatter(rows, [r16, cols[cidx]], v * nv)
                return carry2

            lax.fori_loop(0, K // LANES, grp, 0)
            pltpu.sync_copy(rows, acc.at[dst2d.at[i]], add=True)
            return carry

        lax.fori_loop(0, NCH_MP, chunk, 0)
        plsc.subcore_barrier()
        pltpu.sync_copy(acc.at[pl.ds(s * RPT, RPT)],
                        out.at[c, pl.ds(s * RPT, RPT)])

    return pl.kernel(
        body,
        out_type=jax.ShapeDtypeStruct((NC, N_ACC, D), jnp.float32),
        mesh=plsc.VectorSubcoreMesh(**_MESH),
        compiler_params=_SC_PARAMS,
        scratch_types=[
            pltpu.VMEM((NCH_MP, K), jnp.int32),
            pltpu.VMEM((NCH_MP, K), jnp.int32),
            pltpu.VMEM((NCH_MP, K), jnp.float32),
            pltpu.VMEM((K, D), jnp.float32),
            pltpu.VMEM_SHARED((N_ACC, D), jnp.float32),
            pltpu.SemaphoreType.DMA,
        ],
    )


def _build_gat():
    D = 128
    G = D // LANES

    def body(h0, h1, h2, h3, wgtg, srcg, dstg, zrows, zrows16,
             num, den, src2d, dst2d, w2d, rows, wbuf, dacc, acc, sem):
        c = lax.axis_index("c")
        s = lax.axis_index("s")
        wid = s * NC + c
        pltpu.sync_copy(srcg.at[wid], src2d)
        pltpu.sync_copy(dstg.at[wid], dst2d)
        pltpu.sync_copy(wgtg.at[wid], w2d)
        for r in range(K):
            wbuf[r, :] = jnp.zeros((LANES,), jnp.float32)
        pltpu.sync_copy(zrows16, dacc.at[pl.ds(s * RPT, RPT)])

        zero16 = jnp.zeros((LANES,), jnp.int32)
        iota = _iota16()
        cols = [jnp.full((LANES,), cidx, jnp.int32) for cidx in range(D)]

        for cc, h in enumerate((h0, h1, h2, h3)):
            pltpu.sync_copy(zrows, acc.at[pl.ds(s * RPT, RPT)])
            plsc.subcore_barrier()

            def chunk(i, carry, h=h, cc=cc):
                pltpu.async_copy(h.at[src2d.at[i]], rows, sem).wait()

                def grp(j, carry2):
                    wv = w2d[i, pl.ds(j * LANES, LANES)]
                    r16 = j * LANES + iota
                    if cc == 0:
                        plsc.store_scatter(wbuf, [r16, zero16], wv)
                    for cidx in range(D):
                        v = plsc.load_gather(rows, [r16, cols[cidx]])
                        plsc.store_scatter(rows, [r16, cols[cidx]], v * wv)
                    return carry2

                lax.fori_loop(0, K // LANES, grp, 0)
                pltpu.sync_copy(rows, acc.at[dst2d.at[i]], add=True)
                if cc == 0:
                    pltpu.sync_copy(wbuf, dacc.at[dst2d.at[i]], add=True)
                return carry

            lax.fori_loop(0, NCH_MP, chunk, 0)
            plsc.subcore_barrier()
            pltpu.sync_copy(acc.at[pl.ds(s * RPT, RPT)],
                            num.at[c, cc, pl.ds(s * RPT, RPT)])
            if cc == 0:
                pltpu.sync_copy(dacc.at[pl.ds(s * RPT, RPT)],
                                den.at[c, pl.ds(s * RPT, RPT)])
            plsc.subcore_barrier()

    return pl.kernel(
        body,
        out_type=(
            jax.ShapeDtypeStruct((NC, 4, N_ACC, D), jnp.float32),
            jax.ShapeDtypeStruct((NC, N_ACC, 16), jnp.float32),
        ),
        mesh=plsc.VectorSubcoreMesh(**_MESH),
        compiler_params=_SC_PARAMS,
        scratch_types=[
            pltpu.VMEM((NCH_MP, K), jnp.int32),
            pltpu.VMEM((NCH_MP, K), jnp.int32),
            pltpu.VMEM((NCH_MP, K), jnp.float32),
            pltpu.VMEM((K, D), jnp.float32),
            pltpu.VMEM((K, 16), jnp.float32),
            pltpu.VMEM_SHARED((N_ACC, 16), jnp.float32),
            pltpu.VMEM_SHARED((N_ACC, D), jnp.float32),
            pltpu.SemaphoreType.DMA,
        ],
    )


def _build_final():
    D = 128
    G = D // LANES

    def body(uh, vh, ch, srcg, dstg, b2h, out, src2d, dst2d, gu, gv, cb,
             zrow, b2v, sem):
        c = lax.axis_index("c")
        s = lax.axis_index("s")
        wid = s * NC + c
        base = wid * PW_F
        pltpu.sync_copy(srcg.at[wid], src2d)
        pltpu.sync_copy(dstg.at[wid], dst2d)
        pltpu.sync_copy(b2h, b2v)
        iota = _iota16()
        cols = [jnp.full((LANES,), cidx, jnp.int32) for cidx in range(D)]

        def chunk(i, carry):
            cpu = pltpu.async_copy(uh.at[src2d.at[i]], gu, sem)
            cpv = pltpu.async_copy(vh.at[dst2d.at[i]], gv, sem)
            cpc = pltpu.async_copy(ch.at[pl.ds(base + i * K, K)], cb, sem)
            cpu.wait()
            cpv.wait()
            cpc.wait()

            def grp(j, carry2):
                r16 = j * LANES + iota
                zacc = jnp.zeros((LANES,), jnp.float32)
                for cidx in range(D):
                    t = (plsc.load_gather(gu, [r16, cols[cidx]])
                         + plsc.load_gather(gv, [r16, cols[cidx]]))
                    t = jnp.maximum(t, 0.0)
                    zacc = zacc + t * plsc.load_gather(cb, [r16, cols[cidx]])
                zrow[pl.ds(i * K + j * LANES, LANES)] = zacc
                return carry2

            lax.fori_loop(0, K // LANES, grp, 0)
            return carry

        lax.fori_loop(0, NCH_F, chunk, 0)
        bv = b2v[:]

        def sig(t, carry):
            sl = pl.ds(t * LANES, LANES)
            z = zrow[sl] + bv
            zrow[sl] = 1.0 / (1.0 + jnp.exp(-z))
            return carry

        lax.fori_loop(0, PW_F // LANES, sig, 0)
        pltpu.sync_copy(zrow, out.at[pl.ds(base, PW_F)])

    return pl.kernel(
        body,
        out_type=jax.ShapeDtypeStruct((E_PAD,), jnp.float32),
        mesh=plsc.VectorSubcoreMesh(**_MESH),
        compiler_params=_SC_PARAMS,
        scratch_types=[
            pltpu.VMEM((NCH_F, K), jnp.int32),
            pltpu.VMEM((NCH_F, K), jnp.int32),
            pltpu.VMEM((K, D), jnp.float32),
            pltpu.VMEM((K, D), jnp.float32),
            pltpu.VMEM((K, D), jnp.float32),
            pltpu.VMEM((PW_F,), jnp.float32),
            pltpu.VMEM((LANES,), jnp.float32),
            pltpu.SemaphoreType.DMA,
        ],
    )


def _tc(body, out_shape, grid=None, in_specs=None, out_specs=None):
    kwargs = {}
    if grid is not None:
        kwargs = dict(grid=grid, in_specs=in_specs, out_specs=out_specs)
    return pl.pallas_call(body, out_shape=out_shape, **kwargs)


def _t_dinv(degp):
    def body(p_ref, o_ref):
        deg = p_ref[0] + p_ref[1]
        o_ref[...] = lax.rsqrt(jnp.maximum(deg, 1.0))

    return _tc(body, jax.ShapeDtypeStruct((N_ACC, 16), jnp.float32))(degp)


def _t_comb(p, b, W, dout):
    def body(p_ref, b_ref, w_ref, o_ref):
        x = jnp.tanh(p_ref[0] + p_ref[1] + b_ref[...])
        o_ref[...] = jnp.dot(x, w_ref[...],
                             preferred_element_type=jnp.float32)

    return _tc(body, jax.ShapeDtypeStruct((N_ACC, dout), jnp.float32))(
        p, b.reshape(1, -1), W)


def _t_prep(p, b4, Wg, att_s, att_d):
    def body(p_ref, b_ref, wg_ref, as_ref, ad_ref, h0, h1, h2, h3, av, bv):
        x = jnp.tanh(p_ref[0] + p_ref[1] + b_ref[...])
        hg = jnp.dot(x, wg_ref[...], preferred_element_type=jnp.float32)
        h0[...] = hg[:, 0:128]
        h1[...] = hg[:, 128:256]
        h2[...] = hg[:, 256:384]
        h3[...] = hg[:, 384:512]
        av[...] = jnp.dot(hg, as_ref[...],
                          preferred_element_type=jnp.float32)
        bv[...] = jnp.dot(hg, ad_ref[...],
                          preferred_element_type=jnp.float32)

    outs = tuple([jax.ShapeDtypeStruct((N_ACC, 128), jnp.float32)] * 4
                 + [jax.ShapeDtypeStruct((N_ACC, 1), jnp.float32)] * 2)
    return _tc(body, outs)(p, b4.reshape(1, -1), Wg,
                           att_s.reshape(-1, 1), att_d.reshape(-1, 1))


def _t_gatpost(num, den, bg, Wl1, bl1):
    NB = 1264
    steps = N_ACC // NB

    def body(n_ref, d_ref, bg_ref, wa_ref, wb_ref, b1_ref, u_ref, v_ref):
        n = n_ref[0] + n_ref[1]
        x = jnp.concatenate([n[0], n[1], n[2], n[3]], axis=1)
        d = d_ref[0, :, 0:1] + d_ref[1, :, 0:1]
        x = jnp.maximum(x / jnp.maximum(d, 1e-16) + bg_ref[...], 0.0)
        u_ref[...] = (jnp.dot(x, wa_ref[...],
                              preferred_element_type=jnp.float32)
                      + b1_ref[...])
        v_ref[...] = jnp.dot(x, wb_ref[...],
                             preferred_element_type=jnp.float32)

    grid = (steps,)
    in_specs = [
        pl.BlockSpec((NC, 4, NB, 128), lambda i: (0, 0, i, 0)),
        pl.BlockSpec((NC, NB, 16), lambda i: (0, i, 0)),
        pl.BlockSpec((1, 512), lambda i: (0, 0)),
        pl.BlockSpec((512, 128), lambda i: (0, 0)),
        pl.BlockSpec((512, 128), lambda i: (0, 0)),
        pl.BlockSpec((1, 128), lambda i: (0, 0)),
    ]
    out_specs = [
        pl.BlockSpec((NB, 128), lambda i: (i, 0)),
        pl.BlockSpec((NB, 128), lambda i: (i, 0)),
    ]
    outs = [jax.ShapeDtypeStruct((N_ACC, 128), jnp.float32)] * 2
    return _tc(body, outs, grid, in_specs, out_specs)(
        num, den, bg.reshape(1, -1), Wl1[:512], Wl1[512:],
        bl1.reshape(1, -1))


def _t_cmask(maskf, Wl2):
    NB = 8192
    steps = E_PAD // NB

    def body(m_ref, w_ref, o_ref):
        o_ref[...] = m_ref[...] * 2.0 * w_ref[...]

    grid = (steps,)
    in_specs = [
        pl.BlockSpec((NB, 128), lambda i: (i, 0)),
        pl.BlockSpec((1, 128), lambda i: (0, 0)),
    ]
    out_specs = pl.BlockSpec((NB, 128), lambda i: (i, 0))
    return _tc(body, jax.ShapeDtypeStruct((E_PAD, 128), jnp.float32),
               grid, in_specs, out_specs)(maskf, Wl2.reshape(1, -1))


_sc_deg = _build_deg()
_sc_norm = _build_norm()
_sc_gcn32 = _build_gcn(32)
_sc_gcn64 = _build_gcn(64)
_sc_wgt = _build_wgt()
_sc_gat = _build_gat()
_sc_final = _build_final()


def _pad1(x, n, val=0):
    return jnp.concatenate(
        [x, jnp.full((n - x.shape[0],), val, x.dtype)])


def kernel(edge_index, W1, b1, W2, b2, W3, b3, W4, b4, Wg, att_s, att_d,
           bg, Wl1, bl1, Wl2, bl2):
    ei = edge_index.astype(jnp.int32)
    loop = jnp.arange(N_NODES, dtype=jnp.int32)
    src = jnp.concatenate([ei[0], loop])
    dst = jnp.concatenate([ei[1], loop])
    srcg = _pad1(src, EP_PAD, 0).reshape(NW, NCH_MP, K)
    dstg = _pad1(dst, EP_PAD, TRASH).reshape(NW, NCH_MP, K)
    srcf = _pad1(ei[0], E_PAD, 0).reshape(NW, NCH_F, K)
    dstf = _pad1(ei[1], E_PAD, 0).reshape(NW, NCH_F, K)

    z16 = jnp.zeros((RPT, 16), jnp.float32)
    z32 = jnp.zeros((RPT, 32), jnp.float32)
    z64 = jnp.zeros((RPT, 64), jnp.float32)
    z128 = jnp.zeros((RPT, 128), jnp.float32)
    ones16 = jnp.zeros((K, 16), jnp.float32).at[:, 0].set(1.0)

    degp = _sc_deg(dstg, ones16, z16)
    dinvp = _t_dinv(degp)[:, 0]
    normg = _sc_norm(srcg, dstg, dinvp)

    p = _sc_gcn32(W1, srcg, dstg, normg, z32)
    h = _t_comb(p, b1, W2, 64)
    p = _sc_gcn64(h, srcg, dstg, normg, z64)
    h = _t_comb(p, b2, W3, 64)
    p = _sc_gcn64(h, srcg, dstg, normg, z64)
    h = _t_comb(p, b3, W4, 32)
    p = _sc_gcn32(h, srcg, dstg, normg, z32)

    h0, h1, h2, h3, a_s, a_d = _t_prep(p, b4, Wg, att_s, att_d)
    wgtg = _sc_wgt(srcg, dstg, a_s[:, 0], a_d[:, 0])
    num, den = _sc_gat(h0, h1, h2, h3, wgtg, srcg, dstg, z128, z16)

    u, v = _t_gatpost(num, den, bg, Wl1, bl1)

    mask = jax.random.bernoulli(jax.random.key(42), 0.5,
                                (N_EDGES, 128)).astype(jnp.float32)
    maskp = jnp.concatenate(
        [mask, jnp.zeros((E_PAD - N_EDGES, 128), jnp.float32)])
    cmat = _t_cmask(maskp, Wl2[:, 0])

    b16 = jnp.broadcast_to(bl2, (LANES,)).astype(jnp.float32)
    outp = _sc_final(u, v, cmat, srcf, dstf, b16)
    return outp[:N_EDGES]


# speedup vs baseline: 3.2998x; 3.2998x over previous
import jax
import jax.numpy as jnp
from jax import lax
from jax.experimental import pallas as pl
from jax.experimental.pallas import tpu as pltpu
from jax.experimental.pallas import tpu_sc as plsc

N_NODES = 10000
N_EDGES = 160000
NC, NS, LANES = 2, 16, 16
NW = NC * NS
K = 128

NCH_MP = 42
PW_MP = NCH_MP * K
EP_PAD = NW * PW_MP

NCH_F = 40
PW_F = NCH_F * K
E_PAD = NW * PW_F

N_ACC = 10112
RPT = N_ACC // NS
TRASH = N_NODES + 8

_MESH = dict(core_axis_name="c", subcore_axis_name="s")
_SC_PARAMS = pltpu.CompilerParams(needs_layout_passes=False,
                                  use_tc_tiling_on_sc=False)


def _iota16():
    return lax.broadcasted_iota(jnp.int32, (LANES,), 0)


def _bcast(v, rr):
    return lax.gather(
        v, jnp.full((LANES, 1), rr, jnp.int32),
        lax.GatherDimensionNumbers(offset_dims=(), collapsed_slice_dims=(0,),
                                   start_index_map=(0,)),
        (1,), mode=lax.GatherScatterMode.PROMISE_IN_BOUNDS)


def _build_deg():
    def body(dstg, onesh, zrows, out, dst2d, onesb, dacc):
        c = lax.axis_index("c")
        s = lax.axis_index("s")
        wid = s * NC + c
        pltpu.sync_copy(dstg.at[wid], dst2d)
        pltpu.sync_copy(onesh, onesb)
        pltpu.sync_copy(zrows, dacc.at[pl.ds(s * RPT, RPT)])
        plsc.subcore_barrier()

        def chunk(i, carry):
            pltpu.sync_copy(onesb, dacc.at[dst2d.at[i]], add=True)
            return carry

        lax.fori_loop(0, NCH_MP, chunk, 0)
        plsc.subcore_barrier()
        pltpu.sync_copy(dacc.at[pl.ds(s * RPT, RPT)],
                        out.at[c, pl.ds(s * RPT, RPT)])

    return pl.kernel(
        body,
        out_type=jax.ShapeDtypeStruct((NC, N_ACC, 16), jnp.float32),
        mesh=plsc.VectorSubcoreMesh(**_MESH),
        compiler_params=_SC_PARAMS,
        scratch_types=[
            pltpu.VMEM((NCH_MP, K), jnp.int32),
            pltpu.VMEM((K, 16), jnp.float32),
            pltpu.VMEM_SHARED((N_ACC, 16), jnp.float32),
        ],
    )


def _build_norm():
    def body(srcg, dstg, dinvh, out, src2d, dst2d, dinvv, norm2d):
        c = lax.axis_index("c")
        s = lax.axis_index("s")
        wid = s * NC + c
        pltpu.sync_copy(srcg.at[wid], src2d)
        pltpu.sync_copy(dstg.at[wid], dst2d)
        pltpu.sync_copy(dinvh, dinvv)

        def chunk(i, carry):
            for j in range(K // LANES):
                s16 = src2d[i, pl.ds(j * LANES, LANES)]
                d16 = dst2d[i, pl.ds(j * LANES, LANES)]
                a = plsc.load_gather(dinvv, [s16])
                b = plsc.load_gather(dinvv, [d16])
                norm2d[i, pl.ds(j * LANES, LANES)] = a * b
            return carry

        lax.fori_loop(0, NCH_MP, chunk, 0)
        pltpu.sync_copy(norm2d, out.at[wid])

    return pl.kernel(
        body,
        out_type=jax.ShapeDtypeStruct((NW, NCH_MP, K), jnp.float32),
        mesh=plsc.VectorSubcoreMesh(**_MESH),
        compiler_params=_SC_PARAMS,
        scratch_types=[
            pltpu.VMEM((NCH_MP, K), jnp.int32),
            pltpu.VMEM((NCH_MP, K), jnp.int32),
            pltpu.VMEM((N_ACC,), jnp.float32),
            pltpu.VMEM((NCH_MP, K), jnp.float32),
        ],
    )


def _build_wgt():
    def body(srcg, dstg, ash, adh, out, src2d, dst2d, asv, adv, w2d):
        c = lax.axis_index("c")
        s = lax.axis_index("s")
        wid = s * NC + c
        pltpu.sync_copy(srcg.at[wid], src2d)
        pltpu.sync_copy(dstg.at[wid], dst2d)
        pltpu.sync_copy(ash, asv)
        pltpu.sync_copy(adh, adv)

        def chunk(i, carry):
            for j in range(K // LANES):
                s16 = src2d[i, pl.ds(j * LANES, LANES)]
                d16 = dst2d[i, pl.ds(j * LANES, LANES)]
                e = (plsc.load_gather(asv, [s16])
                     + plsc.load_gather(adv, [d16]))
                e = jnp.maximum(e, e * 0.2)
                w2d[i, pl.ds(j * LANES, LANES)] = jnp.exp(e)
            return carry

        lax.fori_loop(0, NCH_MP, chunk, 0)
        pltpu.sync_copy(w2d, out.at[wid])

    return pl.kernel(
        body,
        out_type=jax.ShapeDtypeStruct((NW, NCH_MP, K), jnp.float32),
        mesh=plsc.VectorSubcoreMesh(**_MESH),
        compiler_params=_SC_PARAMS,
        scratch_types=[
            pltpu.VMEM((NCH_MP, K), jnp.int32),
            pltpu.VMEM((NCH_MP, K), jnp.int32),
            pltpu.VMEM((N_ACC,), jnp.float32),
            pltpu.VMEM((N_ACC,), jnp.float32),
            pltpu.VMEM((NCH_MP, K), jnp.float32),
        ],
    )


def _build_gcn(D):
    G = D // LANES

    def body(h, srcg, dstg, normg, zrows, out, src2d, dst2d, norm2d, rows,
             acc, sem):
        c = lax.axis_index("c")
        s = lax.axis_index("s")
        wid = s * NC + c
        pltpu.sync_copy(srcg.at[wid], src2d)
        pltpu.sync_copy(dstg.at[wid], dst2d)
        pltpu.sync_copy(normg.at[wid], norm2d)
        pltpu.sync_copy(zrows, acc.at[pl.ds(s * RPT, RPT)])
        plsc.subcore_barrier()

        def chunk(i, carry):
            pltpu.async_copy(h.at[src2d.at[i]], rows, sem).wait()

            def grp(j, carry2):
                nv = norm2d[i, pl.ds(j * LANES, LANES)]
                for rr in range(LANES):
                    n16 = _bcast(nv, rr)
                    r = j * LANES + rr
                    for g in range(G):
                        sl = pl.ds(g * LANES, LANES)
                        rows[r, sl] = rows[r, sl] * n16
                return carry2

            lax.fori_loop(0, K // LANES, grp, 0)
            pltpu.sync_copy(rows, acc.at[dst2d.at[i]], add=True)
            return carry

        lax.fori_loop(0, NCH_MP, chunk, 0)
        plsc.subcore_barrier()
        pltpu.sync_copy(acc.at[pl.ds(s * RPT, RPT)],
                        out.at[c, pl.ds(s * RPT, RPT)])

    return pl.kernel(
        body,
        out_type=jax.ShapeDtypeStruct((NC, N_ACC, D), jnp.float32),
        mesh=plsc.VectorSubcoreMesh(**_MESH),
        compiler_params=_SC_PARAMS,
        scratch_types=[
            pltpu.VMEM((NCH_MP, K), jnp.int32),
            pltpu.VMEM((NCH_MP, K), jnp.int32),
            pltpu.VMEM((NCH_MP, K), jnp.float32),
            pltpu.VMEM((K, D), jnp.float32),
            pltpu.VMEM_SHARED((N_ACC, D), jnp.float32),
            pltpu.SemaphoreType.DMA,
        ],
    )


def _build_gat():
    D = 128
    G = D // LANES

    def body(h0, h1, h2, h3, wgtg, srcg, dstg, zrows, zrows16,
             num, den, src2d, dst2d, w2d, rows, wbuf, dacc, acc, sem):
        c = lax.axis_index("c")
        s = lax.axis_index("s")
        wid = s * NC + c
        pltpu.sync_copy(srcg.at[wid], src2d)
        pltpu.sync_copy(dstg.at[wid], dst2d)
        pltpu.sync_copy(wgtg.at[wid], w2d)
        for r in range(K):
            wbuf[r, :] = jnp.zeros((LANES,), jnp.float32)
        pltpu.sync_copy(zrows16, dacc.at[pl.ds(s * RPT, RPT)])

        zero16 = jnp.zeros((LANES,), jnp.int32)
        iota = _iota16()

        for cc, h in enumerate((h0, h1, h2, h3)):
            pltpu.sync_copy(zrows, acc.at[pl.ds(s * RPT, RPT)])
            plsc.subcore_barrier()

            def chunk(i, carry, h=h, cc=cc):
                pltpu.async_copy(h.at[src2d.at[i]], rows, sem).wait()

                def grp(j, carry2):
                    wv = w2d[i, pl.ds(j * LANES, LANES)]
                    if cc == 0:
                        plsc.store_scatter(wbuf, [j * LANES + iota, zero16],
                                           wv)
                    for rr in range(LANES):
                        n16 = _bcast(wv, rr)
                        r = j * LANES + rr
                        for g in range(G):
                            sl = pl.ds(g * LANES, LANES)
                            rows[r, sl] = rows[r, sl] * n16
                    return carry2

                lax.fori_loop(0, K // LANES, grp, 0)
                pltpu.sync_copy(rows, acc.at[dst2d.at[i]], add=True)
                if cc == 0:
                    pltpu.sync_copy(wbuf, dacc.at[dst2d.at[i]], add=True)
                return carry

            lax.fori_loop(0, NCH_MP, chunk, 0)
            plsc.subcore_barrier()
            pltpu.sync_copy(acc.at[pl.ds(s * RPT, RPT)],
                            num.at[c, cc, pl.ds(s * RPT, RPT)])
            if cc == 0:
                pltpu.sync_copy(dacc.at[pl.ds(s * RPT, RPT)],
                                den.at[c, pl.ds(s * RPT, RPT)])
            plsc.subcore_barrier()

    return pl.kernel(
        body,
        out_type=(
            jax.ShapeDtypeStruct((NC, 4, N_ACC, D), jnp.float32),
            jax.ShapeDtypeStruct((NC, N_ACC, 16), jnp.float32),
        ),
        mesh=plsc.VectorSubcoreMesh(**_MESH),
        compiler_params=_SC_PARAMS,
        scratch_types=[
            pltpu.VMEM((NCH_MP, K), jnp.int32),
            pltpu.VMEM((NCH_MP, K), jnp.int32),
            pltpu.VMEM((NCH_MP, K), jnp.float32),
            pltpu.VMEM((K, D), jnp.float32),
            pltpu.VMEM((K, 16), jnp.float32),
            pltpu.VMEM_SHARED((N_ACC, 16), jnp.float32),
            pltpu.VMEM_SHARED((N_ACC, D), jnp.float32),
            pltpu.SemaphoreType.DMA,
        ],
    )


def _build_final():
    D = 128
    G = D // LANES

    def body(uh, vh, ch, srcg, dstg, b2h, out, src2d, dst2d, gu, gv, cb,
             zrow, b2v, sem):
        c = lax.axis_index("c")
        s = lax.axis_index("s")
        wid = s * NC + c
        base = wid * PW_F
        pltpu.sync_copy(srcg.at[wid], src2d)
        pltpu.sync_copy(dstg.at[wid], dst2d)
        pltpu.sync_copy(b2h, b2v)
        iota = _iota16()
        zeroi = jnp.zeros((LANES,), jnp.int32)
        lastlane = iota == (LANES - 1)

        def chunk(i, carry):
            cpu = pltpu.async_copy(uh.at[src2d.at[i]], gu, sem)
            cpv = pltpu.async_copy(vh.at[dst2d.at[i]], gv, sem)
            cpc = pltpu.async_copy(ch.at[pl.ds(base + i * K, K)], cb, sem)
            cpu.wait()
            cpv.wait()
            cpc.wait()

            def grp(j, carry2):
                for rr in range(LANES):
                    r = j * LANES + rr
                    acc = jnp.zeros((LANES,), jnp.float32)
                    for g in range(G):
                        sl = pl.ds(g * LANES, LANES)
                        t = jnp.maximum(gu[r, sl] + gv[r, sl], 0.0)
                        acc = acc + t * cb[r, sl]
                    cs = plsc.cumsum(acc)
                    plsc.store_scatter(zrow, [i * K + r + zeroi],
                                       cs, mask=lastlane)
                return carry2

            lax.fori_loop(0, K // LANES, grp, 0)
            return carry

        lax.fori_loop(0, NCH_F, chunk, 0)
        bv = b2v[:]

        def sig(t, carry):
            sl = pl.ds(t * LANES, LANES)
            z = zrow[sl] + bv
            zrow[sl] = 1.0 / (1.0 + jnp.exp(-z))
            return carry

        lax.fori_loop(0, PW_F // LANES, sig, 0)
        pltpu.sync_copy(zrow, out.at[pl.ds(base, PW_F)])

    return pl.kernel(
        body,
        out_type=jax.ShapeDtypeStruct((E_PAD,), jnp.float32),
        mesh=plsc.VectorSubcoreMesh(**_MESH),
        compiler_params=_SC_PARAMS,
        scratch_types=[
            pltpu.VMEM((NCH_F, K), jnp.int32),
            pltpu.VMEM((NCH_F, K), jnp.int32),
            pltpu.VMEM((K, D), jnp.float32),
            pltpu.VMEM((K, D), jnp.float32),
            pltpu.VMEM((K, D), jnp.float32),
            pltpu.VMEM((PW_F,), jnp.float32),
            pltpu.VMEM((LANES,), jnp.float32),
            pltpu.SemaphoreType.DMA,
        ],
    )


def _tc(body, out_shape, grid=None, in_specs=None, out_specs=None):
    kwargs = {}
    if grid is not None:
        kwargs = dict(grid=grid, in_specs=in_specs, out_specs=out_specs)
    return pl.pallas_call(body, out_shape=out_shape, **kwargs)


def _t_dinv(degp):
    def body(p_ref, o_ref):
        deg = p_ref[0] + p_ref[1]
        o_ref[...] = lax.rsqrt(jnp.maximum(deg, 1.0))

    return _tc(body, jax.ShapeDtypeStruct((N_ACC, 16), jnp.float32))(degp)


def _t_comb(p, b, W, dout):
    def body(p_ref, b_ref, w_ref, o_ref):
        x = jnp.tanh(p_ref[0] + p_ref[1] + b_ref[...])
        o_ref[...] = jnp.dot(x, w_ref[...],
                             preferred_element_type=jnp.float32)

    return _tc(body, jax.ShapeDtypeStruct((N_ACC, dout), jnp.float32))(
        p, b.reshape(1, -1), W)


def _t_prep(p, b4, Wg, att_s, att_d):
    def body(p_ref, b_ref, wg_ref, as_ref, ad_ref, h0, h1, h2, h3, av, bv):
        x = jnp.tanh(p_ref[0] + p_ref[1] + b_ref[...])
        hg = jnp.dot(x, wg_ref[...], preferred_element_type=jnp.float32)
        h0[...] = hg[:, 0:128]
        h1[...] = hg[:, 128:256]
        h2[...] = hg[:, 256:384]
        h3[...] = hg[:, 384:512]
        av[...] = jnp.dot(hg, as_ref[...],
                          preferred_element_type=jnp.float32)
        bv[...] = jnp.dot(hg, ad_ref[...],
                          preferred_element_type=jnp.float32)

    outs = tuple([jax.ShapeDtypeStruct((N_ACC, 128), jnp.float32)] * 4
                 + [jax.ShapeDtypeStruct((N_ACC, 1), jnp.float32)] * 2)
    return _tc(body, outs)(p, b4.reshape(1, -1), Wg,
                           att_s.reshape(-1, 1), att_d.reshape(-1, 1))


def _t_gatpost(num, den, bg, Wl1, bl1):
    NB = 1264
    steps = N_ACC // NB

    def body(n_ref, d_ref, bg_ref, wa_ref, wb_ref, b1_ref, u_ref, v_ref):
        n = n_ref[0] + n_ref[1]
        x = jnp.concatenate([n[0], n[1], n[2], n[3]], axis=1)
        d = d_ref[0, :, 0:1] + d_ref[1, :, 0:1]
        x = jnp.maximum(x / jnp.maximum(d, 1e-16) + bg_ref[...], 0.0)
        u_ref[...] = (jnp.dot(x, wa_ref[...],
                              preferred_element_type=jnp.float32)
                      + b1_ref[...])
        v_ref[...] = jnp.dot(x, wb_ref[...],
                             preferred_element_type=jnp.float32)

    grid = (steps,)
    in_specs = [
        pl.BlockSpec((NC, 4, NB, 128), lambda i: (0, 0, i, 0)),
        pl.BlockSpec((NC, NB, 16), lambda i: (0, i, 0)),
        pl.BlockSpec((1, 512), lambda i: (0, 0)),
        pl.BlockSpec((512, 128), lambda i: (0, 0)),
        pl.BlockSpec((512, 128), lambda i: (0, 0)),
        pl.BlockSpec((1, 128), lambda i: (0, 0)),
    ]
    out_specs = [
        pl.BlockSpec((NB, 128), lambda i: (i, 0)),
        pl.BlockSpec((NB, 128), lambda i: (i, 0)),
    ]
    outs = [jax.ShapeDtypeStruct((N_ACC, 128), jnp.float32)] * 2
    return _tc(body, outs, grid, in_specs, out_specs)(
        num, den, bg.reshape(1, -1), Wl1[:512], Wl1[512:],
        bl1.reshape(1, -1))


def _t_cmask(maskf, Wl2):
    NB = 8192
    steps = E_PAD // NB

    def body(m_ref, w_ref, o_ref):
        o_ref[...] = m_ref[...] * 2.0 * w_ref[...]

    grid = (steps,)
    in_specs = [
        pl.BlockSpec((NB, 128), lambda i: (i, 0)),
        pl.BlockSpec((1, 128), lambda i: (0, 0)),
    ]
    out_specs = pl.BlockSpec((NB, 128), lambda i: (i, 0))
    return _tc(body, jax.ShapeDtypeStruct((E_PAD, 128), jnp.float32),
               grid, in_specs, out_specs)(maskf, Wl2.reshape(1, -1))


_sc_deg = _build_deg()
_sc_norm = _build_norm()
_sc_gcn32 = _build_gcn(32)
_sc_gcn64 = _build_gcn(64)
_sc_wgt = _build_wgt()
_sc_gat = _build_gat()
_sc_final = _build_final()


def _pad1(x, n, val=0):
    return jnp.concatenate(
        [x, jnp.full((n - x.shape[0],), val, x.dtype)])


def kernel(edge_index, W1, b1, W2, b2, W3, b3, W4, b4, Wg, att_s, att_d,
           bg, Wl1, bl1, Wl2, bl2):
    ei = edge_index.astype(jnp.int32)
    loop = jnp.arange(N_NODES, dtype=jnp.int32)
    src = jnp.concatenate([ei[0], loop])
    dst = jnp.concatenate([ei[1], loop])
    srcg = _pad1(src, EP_PAD, 0).reshape(NW, NCH_MP, K)
    dstg = _pad1(dst, EP_PAD, TRASH).reshape(NW, NCH_MP, K)
    srcf = _pad1(ei[0], E_PAD, 0).reshape(NW, NCH_F, K)
    dstf = _pad1(ei[1], E_PAD, 0).reshape(NW, NCH_F, K)

    z16 = jnp.zeros((RPT, 16), jnp.float32)
    z32 = jnp.zeros((RPT, 32), jnp.float32)
    z64 = jnp.zeros((RPT, 64), jnp.float32)
    z128 = jnp.zeros((RPT, 128), jnp.float32)
    ones16 = jnp.zeros((K, 16), jnp.float32).at[:, 0].set(1.0)

    degp = _sc_deg(dstg, ones16, z16)
    dinvp = _t_dinv(degp)[:, 0]
    normg = _sc_norm(srcg, dstg, dinvp)

    p = _sc_gcn32(W1, srcg, dstg, normg, z32)
    h = _t_comb(p, b1, W2, 64)
    p = _sc_gcn64(h, srcg, dstg, normg, z64)
    h = _t_comb(p, b2, W3, 64)
    p = _sc_gcn64(h, srcg, dstg, normg, z64)
    h = _t_comb(p, b3, W4, 32)
    p = _sc_gcn32(h, srcg, dstg, normg, z32)

    h0, h1, h2, h3, a_s, a_d = _t_prep(p, b4, Wg, att_s, att_d)
    wgtg = _sc_wgt(srcg, dstg, a_s[:, 0], a_d[:, 0])
    num, den = _sc_gat(h0, h1, h2, h3, wgtg, srcg, dstg, z128, z16)

    u, v = _t_gatpost(num, den, bg, Wl1, bl1)

    mask = jax.random.bernoulli(jax.random.key(42), 0.5,
                                (N_EDGES, 128)).astype(jnp.float32)
    maskp = jnp.concatenate(
        [mask, jnp.zeros((E_PAD - N_EDGES, 128), jnp.float32)])
    cmat = _t_cmask(maskp, Wl2[:, 0])

    b16 = jnp.broadcast_to(bl2, (LANES,)).astype(jnp.float32)
    outp = _sc_final(u, v, cmat, srcf, dstf, b16)
    return outp[:N_EDGES]
